# baseline (speedup 1.0000x reference)
import functools

import jax
import jax.numpy as jnp
from jax import lax
from jax.experimental import pallas as pl
from jax.experimental.pallas import tpu as pltpu
from jax.experimental.pallas import tpu_sc as plsc

VOCAB = 100000
HIDDEN = 2048
BATCH = 4
SEQ = 2048
B = BATCH * SEQ

NUM_CORES = 2
NUM_SUBCORES = 16
NW = NUM_CORES * NUM_SUBCORES
BPW = B // NW
CHUNK = 16
NCHUNK = BPW // CHUNK
NBUF = 2
WPB = SEQ // BPW


def _emb_kernel(idx_hbm, table_hbm, out_hbm, idx_v, rows_v, gsem, ssem):
    wid = lax.axis_index("s") * NUM_CORES + lax.axis_index("c")
    b0 = wid // WPB
    col = (wid % WPB) * BPW
    pltpu.sync_copy(idx_hbm.at[b0, pl.ds(col, BPW)], idx_v)

    def issue_g(ch):
        return pltpu.async_copy(
            table_hbm.at[idx_v.at[pl.ds(ch * CHUNK, CHUNK)]],
            rows_v.at[ch % NBUF],
            gsem,
        )

    def issue_s(ch):
        return pltpu.async_copy(
            rows_v.at[ch % NBUF],
            out_hbm.at[b0, pl.ds(col + ch * CHUNK, CHUNK)],
            ssem,
        )

    for p in range(NCHUNK // NBUF):
        gs = [issue_g(p * NBUF + i) for i in range(NBUF)]
        for h in gs:
            h.wait()
        plsc.subcore_barrier()
        ss = [issue_s(p * NBUF + i) for i in range(NBUF)]
        for h in ss:
            h.wait()
        plsc.subcore_barrier()


@jax.jit
def _emb(token_ids, table):
    mesh = plsc.VectorSubcoreMesh(core_axis_name="c", subcore_axis_name="s")
    f = functools.partial(
        pl.kernel,
        mesh=mesh,
        out_type=jax.ShapeDtypeStruct((BATCH, SEQ, HIDDEN), jnp.float32),
        scratch_types=[
            pltpu.VMEM((BPW,), jnp.int32),
            pltpu.VMEM((NBUF, CHUNK, HIDDEN), jnp.float32),
            pltpu.SemaphoreType.DMA,
            pltpu.SemaphoreType.DMA,
        ],
    )(_emb_kernel)
    return f(token_ids, table)


def kernel(token_ids, embed_weight):
    return _emb(token_ids.astype(jnp.int32), embed_weight)


# speedup vs baseline: 1.0928x; 1.0928x over previous
import functools

import jax
import jax.numpy as jnp
from jax import lax
from jax.experimental import pallas as pl
from jax.experimental.pallas import tpu as pltpu
from jax.experimental.pallas import tpu_sc as plsc

VOCAB = 100000
HIDDEN = 2048
BATCH = 4
SEQ = 2048
B = BATCH * SEQ

NUM_CORES = 2
NUM_SUBCORES = 16
NW = NUM_CORES * NUM_SUBCORES
BPW = B // NW
CHUNK = 16
NCHUNK = BPW // CHUNK
NBUF = 3
WPB = SEQ // BPW


def _emb_kernel(idx_hbm, table_hbm, out_hbm, idx_v, rows_v, gsem, ssem):
    wid = lax.axis_index("s") * NUM_CORES + lax.axis_index("c")
    b0 = wid // WPB
    col = (wid % WPB) * BPW
    pltpu.sync_copy(idx_hbm.at[b0, pl.ds(col, BPW)], idx_v)

    def issue_g(ch):
        return pltpu.async_copy(
            table_hbm.at[idx_v.at[pl.ds(ch * CHUNK, CHUNK)]],
            rows_v.at[ch % NBUF],
            gsem,
        )

    def issue_s(ch):
        return pltpu.async_copy(
            rows_v.at[ch % NBUF],
            out_hbm.at[b0, pl.ds(col + ch * CHUNK, CHUNK)],
            ssem,
        )

    g = {0: issue_g(0), 1: issue_g(1)}
    s = {}
    for ch in range(NCHUNK):
        g[ch].wait()
        s[ch] = issue_s(ch)
        nxt = ch + 2
        if nxt < NCHUNK:
            if nxt - NBUF >= 0:
                s[nxt - NBUF].wait()
            g[nxt] = issue_g(nxt)
    for j in range(max(0, NCHUNK - NBUF), NCHUNK):
        s[j].wait()


@jax.jit
def _emb(token_ids, table):
    mesh = plsc.VectorSubcoreMesh(core_axis_name="c", subcore_axis_name="s")
    f = functools.partial(
        pl.kernel,
        mesh=mesh,
        out_type=jax.ShapeDtypeStruct((BATCH, SEQ, HIDDEN), jnp.float32),
        scratch_types=[
            pltpu.VMEM((BPW,), jnp.int32),
            pltpu.VMEM((NBUF, CHUNK, HIDDEN), jnp.float32),
            pltpu.SemaphoreType.DMA,
            pltpu.SemaphoreType.DMA,
        ],
    )(_emb_kernel)
    return f(token_ids, table)


def kernel(token_ids, embed_weight):
    return _emb(token_ids.astype(jnp.int32), embed_weight)
